# baseline (speedup 1.0000x reference)
import functools

import numpy as np
import jax
import jax.numpy as jnp
from jax import lax
from jax.experimental import pallas as pl
from jax.experimental.pallas import tpu as pltpu
from jax.experimental.pallas import tpu_sc as plsc

_VOCAB = 1000
_D = 768
_SEQ = 2048
_BATCH = 4

_NC = 2
_NS = 16
_NW = _NC * _NS
_ROWS = _BATCH * _SEQ
_SPW = _SEQ // _NW
_CH = 32
_NCH = _SPW // _CH
_NU = _NCH * _BATCH
_LANES = 16
_RING = 3


def _positional_table() -> np.ndarray:
    even_i = np.arange(0, _D, 2, dtype=np.float32)
    denominator = np.power(10000.0, even_i / np.float32(_D)).astype(np.float32)
    position = np.arange(_SEQ, dtype=np.float32).reshape(_SEQ, 1)
    even_pe = np.sin(position / denominator)
    odd_pe = np.cos(position / denominator)
    stacked = np.stack([even_pe, odd_pe], axis=2)
    return stacked.reshape(_SEQ, _D).astype(np.float32)


def _pack_bf16_pairs(pos: np.ndarray) -> np.ndarray:
    v = pos.astype(jnp.bfloat16).reshape(_SEQ, _D // 32, 2, 16)
    inter = np.ascontiguousarray(v.transpose(0, 1, 3, 2))
    return inter.reshape(_SEQ, _D).view(np.int32)


_POS = _pack_bf16_pairs(_positional_table())

_mesh = plsc.VectorSubcoreMesh(core_axis_name="c", subcore_axis_name="s")


@functools.partial(
    pl.kernel,
    mesh=_mesh,
    out_type=jax.ShapeDtypeStruct((_ROWS, _D), jnp.float32),
    scratch_types=[
        pltpu.VMEM((_BATCH * _SPW,), jnp.int32),
        pltpu.VMEM((_SPW * _D // 2,), jnp.int32),
        pltpu.VMEM((_CH, _D), jnp.float32),
        pltpu.VMEM((_CH, _D), jnp.float32),
        pltpu.VMEM((_CH, _D), jnp.float32),
        pltpu.SemaphoreType.DMA,
        pltpu.SemaphoreType.DMA,
        pltpu.SemaphoreType.DMA,
        pltpu.SemaphoreType.DMA,
        pltpu.SemaphoreType.DMA,
        pltpu.SemaphoreType.DMA,
        pltpu.SemaphoreType.DMA,
    ],
)
def _emb_kernel(x_hbm, table_hbm, pos_hbm, out_hbm,
                idx_v, pos_v, rows0, rows1, rows2,
                psem, gsem0, gsem1, gsem2, ssem0, ssem1, ssem2):
    wid = lax.axis_index("s") * _NC + lax.axis_index("c")
    s_base = wid * _SPW

    rows = (rows0, rows1, rows2)
    gsem = (gsem0, gsem1, gsem2)
    ssem = (ssem0, ssem1, ssem2)

    def unit_rowbase(u):
        c, b = divmod(u, _BATCH)
        return b * _SEQ + s_base + c * _CH

    def unit_idxoff(u):
        c, b = divmod(u, _BATCH)
        return b * _SPW + c * _CH

    pos_dma = pltpu.async_copy(
        pos_hbm.at[pl.ds(s_base * (_D // 2), _SPW * (_D // 2))], pos_v, psem)
    for b in range(_BATCH):
        pltpu.sync_copy(x_hbm.at[pl.ds(b * _SEQ + s_base, _SPW)],
                        idx_v.at[pl.ds(b * _SPW, _SPW)])

    def issue_gather(u):
        return pltpu.async_copy(
            table_hbm.at[idx_v.at[pl.ds(unit_idxoff(u), _CH)]],
            rows[u % _RING], gsem[u % _RING])

    gather_h = {0: issue_gather(0), 1: issue_gather(1)}
    scatter_h = {}

    for u in range(_NU):
        if u + 2 < _NU:
            if u - 1 >= 0:
                scatter_h[u - 1].wait()
            gather_h[u + 2] = issue_gather(u + 2)
        gather_h[u].wait()
        if u == 0:
            pos_dma.wait()

        c = u // _BATCH
        rv = rows[u % _RING]

        @plsc.parallel_loop(0, _CH, step=1, unroll=4)
        def row_body(r, c=c, rv=rv):
            pbase = (c * _CH + r) * (_D // 2)
            for dpart in range(_D // 32):
                w = pos_v[pl.ds(pbase + dpart * 16, _LANES)]
                a = lax.bitcast_convert_type(w << 16, jnp.float32)
                b = lax.bitcast_convert_type(w & jnp.int32(-65536), jnp.float32)
                plsc.addupdate(rv.at[r, pl.ds(dpart * 32, _LANES)], a)
                plsc.addupdate(rv.at[r, pl.ds(dpart * 32 + 16, _LANES)], b)

        scatter_h[u] = pltpu.async_copy(
            rv, out_hbm.at[pl.ds(unit_rowbase(u), _CH)], ssem[u % _RING])

    for u in range(_NU - 2, _NU):
        scatter_h[u].wait()
    scatter_h[_NU - 3].wait()


def kernel(x, table):
    pos = jnp.asarray(_POS).reshape(_SEQ * (_D // 2))
    out = _emb_kernel(x.reshape(_ROWS).astype(jnp.int32), table, pos)
    return out.reshape(_BATCH, _SEQ, _D)


# speedup vs baseline: 1.1308x; 1.1308x over previous
import functools

import numpy as np
import jax
import jax.numpy as jnp
from jax import lax
from jax.experimental import pallas as pl
from jax.experimental.pallas import tpu as pltpu
from jax.experimental.pallas import tpu_sc as plsc

_VOCAB = 1000
_D = 768
_SEQ = 2048
_BATCH = 4

_NC = 2
_NS = 16
_NW = _NC * _NS
_ROWS = _BATCH * _SEQ
_SPW = _SEQ // _NW
_CH = 8
_NCH = _SPW // _CH
_LANES = 16
_NG = 3
_STEP_OFF = _NW * 2 * _D


def _positional_table() -> np.ndarray:
    even_i = np.arange(0, _D, 2, dtype=np.float32)
    denominator = np.power(10000.0, even_i / np.float32(_D)).astype(np.float32)
    position = np.arange(_SEQ, dtype=np.float32).reshape(_SEQ, 1)
    even_pe = np.sin(position / denominator)
    odd_pe = np.cos(position / denominator)
    stacked = np.stack([even_pe, odd_pe], axis=2)
    return stacked.reshape(_SEQ, _D).astype(np.float32)


def _rotation_consts() -> np.ndarray:
    pe = _positional_table()
    base = pe[:: _SPW].copy()
    base_sw = base.reshape(_NW, _D // 2, 2)[:, :, ::-1].reshape(_NW, _D)
    pairs = np.stack([base, base_sw], axis=1)
    p1 = pe[1]
    crow = np.repeat(p1[1::2], 2).astype(np.float32)
    srow = np.empty(_D, np.float32)
    srow[0::2] = p1[0::2]
    srow[1::2] = -p1[0::2]
    return np.concatenate(
        [pairs.reshape(-1), crow, srow]).astype(np.float32)


_CONSTS = _rotation_consts()

_mesh = plsc.VectorSubcoreMesh(core_axis_name="c", subcore_axis_name="s")

_scratch = (
    [pltpu.VMEM((_BATCH * _SPW,), jnp.int32)]
    + [pltpu.VMEM((2 * _D,), jnp.float32)]
    + [pltpu.VMEM((2 * _D,), jnp.float32)]
    + [pltpu.VMEM((_CH, _D), jnp.float32) for _ in range(_NG * _BATCH)]
    + [pltpu.SemaphoreType.DMA for _ in range(2 * _NG * _BATCH)]
)


@functools.partial(
    pl.kernel,
    mesh=_mesh,
    out_type=jax.ShapeDtypeStruct((_ROWS, _D), jnp.float32),
    scratch_types=_scratch,
)
def _emb_kernel(x_hbm, table_hbm, cst_hbm, out_hbm, idx_v, base_v, step_v,
                *scr):
    nslot = _NG * _BATCH
    rows = scr[:nslot]
    gsem = scr[nslot:2 * nslot]
    ssem = scr[2 * nslot:]

    wid = lax.axis_index("s") * _NC + lax.axis_index("c")
    s_base = wid * _SPW

    pltpu.sync_copy(cst_hbm.at[pl.ds(wid * (2 * _D), 2 * _D)], base_v)
    pltpu.sync_copy(cst_hbm.at[pl.ds(_STEP_OFF, 2 * _D)], step_v)
    for b in range(_BATCH):
        pltpu.sync_copy(x_hbm.at[pl.ds(b * _SEQ + s_base, _SPW)],
                        idx_v.at[pl.ds(b * _SPW, _SPW)])

    def slot(c, b):
        return (c % _NG) * _BATCH + b

    def issue_gather(c, b):
        return pltpu.async_copy(
            table_hbm.at[idx_v.at[pl.ds(b * _SPW + c * _CH, _CH)]],
            rows[slot(c, b)], gsem[slot(c, b)])

    def issue_scatter(c, b):
        return pltpu.async_copy(
            rows[slot(c, b)],
            out_hbm.at[pl.ds(b * _SEQ + s_base + c * _CH, _CH)],
            ssem[slot(c, b)])

    gh = {(c, b): issue_gather(c, b) for c in (0, 1) for b in range(_BATCH)}
    sh = {}

    for c in range(_NCH):
        if c + 2 < _NCH:
            for b in range(_BATCH):
                if c - 1 >= 0:
                    sh[(c - 1, b)].wait()
                gh[(c + 2, b)] = issue_gather(c + 2, b)
        for b in range(_BATCH):
            gh[(c, b)].wait()

        cslots = [rows[slot(c, b)] for b in range(_BATCH)]

        def dbody(dpart, carry, cslots=cslots):
            off = pl.multiple_of(dpart * _LANES, _LANES)
            off2 = pl.multiple_of(_D + dpart * _LANES, _LANES)
            cur = base_v[pl.ds(off, _LANES)]
            cw = base_v[pl.ds(off2, _LANES)]
            crot = step_v[pl.ds(off, _LANES)]
            srot = step_v[pl.ds(off2, _LANES)]
            for r in range(_CH):
                for rv in cslots:
                    plsc.addupdate(rv.at[r, pl.ds(off, _LANES)], cur)
                cur, cw = cur * crot + cw * srot, cw * crot - cur * srot
            base_v[pl.ds(off, _LANES)] = cur
            base_v[pl.ds(off2, _LANES)] = cw
            return carry

        lax.fori_loop(0, _D // _LANES, dbody, 0, unroll=False)

        for b in range(_BATCH):
            sh[(c, b)] = issue_scatter(c, b)

    for c in range(_NCH - 3, _NCH):
        for b in range(_BATCH):
            sh[(c, b)].wait()


def kernel(x, table):
    cst = jnp.asarray(_CONSTS)
    out = _emb_kernel(x.reshape(_ROWS).astype(jnp.int32), table, cst)
    return out.reshape(_BATCH, _SEQ, _D)
